# baseline (speedup 1.0000x reference)
import functools

import jax
import jax.numpy as jnp
from jax import lax
from jax.experimental import pallas as pl
from jax.experimental.pallas import tpu as pltpu
from jax.experimental.pallas import tpu_sc as plsc

_INFO = plsc.get_sparse_core_info()
_NC = _INFO.num_cores
_NS = _INFO.num_subcores
_LANES = _INFO.num_lanes
_NW = _NC * _NS

_B = 16384
_E = 16
_BPW = _B // _NW

_MESH = plsc.VectorSubcoreMesh(core_axis_name="c", subcore_axis_name="s")


@functools.partial(
    pl.kernel,
    mesh=_MESH,
    compiler_params=pltpu.CompilerParams(use_tc_tiling_on_sc=False),
    out_type=[
        jax.ShapeDtypeStruct((_NW, _LANES), jnp.float32),
        jax.ShapeDtypeStruct((_B,), jnp.float32),
        jax.ShapeDtypeStruct((_B,), jnp.float32),
    ],
    scratch_types=[
        pltpu.VMEM((_BPW,), jnp.int32),
        pltpu.VMEM((_BPW,), jnp.int32),
        pltpu.VMEM((_BPW, _E), jnp.float32),
        pltpu.VMEM((_BPW, _E), jnp.float32),
        pltpu.VMEM((_BPW,), jnp.float32),
        pltpu.VMEM((_BPW,), jnp.float32),
        pltpu.VMEM((_LANES,), jnp.float32),
        pltpu.SemaphoreType.DMA,
    ],
)
def _sc_gather_dot(uidx_hbm, pidx_hbm, uemb_hbm, pemb_hbm, ubias_hbm,
                   pbias_hbm, partials_hbm, ub_out_hbm, pb_out_hbm,
                   uidx_v, pidx_v, urows_v, prows_v, ub_v, pb_v, acc_v, sem):
    wid = lax.axis_index("s") * _NC + lax.axis_index("c")
    base = wid * _BPW

    pltpu.sync_copy(uidx_hbm.at[pl.ds(base, _BPW)], uidx_v)
    pltpu.sync_copy(pidx_hbm.at[pl.ds(base, _BPW)], pidx_v)

    cu = pltpu.async_copy(uemb_hbm.at[uidx_v], urows_v, sem)
    cp = pltpu.async_copy(pemb_hbm.at[pidx_v], prows_v, sem)
    cub = pltpu.async_copy(ubias_hbm.at[uidx_v], ub_v, sem)
    cpb = pltpu.async_copy(pbias_hbm.at[pidx_v], pb_v, sem)
    cu.wait()
    cp.wait()
    cub.wait()
    cpb.wait()

    def body(i, acc):
        return acc + urows_v[i] * prows_v[i]

    acc = lax.fori_loop(0, _BPW, body, jnp.zeros((_LANES,), jnp.float32),
                        unroll=8)
    acc_v[...] = acc

    pltpu.sync_copy(acc_v, partials_hbm.at[wid])
    pltpu.sync_copy(ub_v, ub_out_hbm.at[pl.ds(base, _BPW)])
    pltpu.sync_copy(pb_v, pb_out_hbm.at[pl.ds(base, _BPW)])


def _tc_combine_body(partials_ref, ub_ref, pb_ref, out_ref):
    s = jnp.sum(partials_ref[...])
    out_ref[...] = jax.nn.sigmoid(ub_ref[...] + pb_ref[...] + s)


def kernel(inputs, user_embedding, user_bias, place_embedding, place_bias):
    uidx = inputs[:, 0].astype(jnp.int32)
    pidx = inputs[:, 1].astype(jnp.int32)
    partials, ubg, pbg = _sc_gather_dot(
        uidx, pidx, user_embedding, place_embedding,
        user_bias.reshape(-1), place_bias.reshape(-1))
    out = pl.pallas_call(
        _tc_combine_body,
        out_shape=jax.ShapeDtypeStruct((128, 128), jnp.float32),
    )(partials.reshape(4, 128), ubg.reshape(128, 128), pbg.reshape(128, 128))
    return out.reshape(_B, 1)
